# Initial kernel scaffold; baseline (speedup 1.0000x reference)
import jax
import jax.numpy as jnp
from jax.experimental import pallas as pl


def kernel(prefix, emb_table):
    raise NotImplementedError("write your pallas kernel here")


# speedup vs baseline: 1.7610x; 1.7610x over previous
import functools

import jax
import jax.numpy as jnp
from jax import lax
from jax.experimental import pallas as pl
from jax.experimental.pallas import tpu as pltpu
from jax.experimental.pallas import tpu_sc as plsc

B = 32
S = 128
V = 128
D = 18432
NB = B * S
NC = 2
NS = 16
NW = NC * NS
BPW = NB // NW
NBUF = 4

_mesh = plsc.VectorSubcoreMesh(core_axis_name="c", subcore_axis_name="s")


@functools.partial(
    pl.kernel,
    out_type=jax.ShapeDtypeStruct((NB, D), jnp.float32),
    mesh=_mesh,
    scratch_types=[
        pltpu.VMEM((BPW, 1), jnp.int32),
        pltpu.VMEM((1, D), jnp.float32),
        pltpu.VMEM((1, D), jnp.float32),
        pltpu.VMEM((1, D), jnp.float32),
        pltpu.VMEM((1, D), jnp.float32),
        pltpu.SemaphoreType.DMA,
        pltpu.SemaphoreType.DMA,
        pltpu.SemaphoreType.DMA,
        pltpu.SemaphoreType.DMA,
        pltpu.SemaphoreType.DMA,
        pltpu.SemaphoreType.DMA,
        pltpu.SemaphoreType.DMA,
        pltpu.SemaphoreType.DMA,
    ],
)
def _sc_gather(idx_hbm, table_hbm, out_hbm, idx_v,
               b0, b1, b2, b3, g0, g1, g2, g3, s0, s1, s2, s3):
    wid = lax.axis_index("s") * NC + lax.axis_index("c")
    base = wid * BPW
    bufs = (b0, b1, b2, b3)
    gsems = (g0, g1, g2, g3)
    ssems = (s0, s1, s2, s3)

    pltpu.sync_copy(idx_hbm.at[wid], idx_v)

    for b in range(NBUF):
        pltpu.async_copy(table_hbm.at[idx_v.at[b]], bufs[b], gsems[b])

    def body(i, carry):
        for b in range(NBUF):
            j = i * NBUF + b
            pltpu.make_async_copy(
                table_hbm.at[idx_v.at[j]], bufs[b], gsems[b]).wait()
            pltpu.async_copy(
                bufs[b], out_hbm.at[pl.ds(base + j, 1)], ssems[b])

            @pl.when(j + NBUF < BPW)
            def _():
                pltpu.make_async_copy(
                    bufs[b], out_hbm.at[pl.ds(base + j, 1)], ssems[b]).wait()
                pltpu.async_copy(
                    table_hbm.at[idx_v.at[j + NBUF]], bufs[b], gsems[b])
        return carry

    lax.fori_loop(0, BPW // NBUF, body, 0)

    for b in range(NBUF):
        j = BPW - NBUF + b
        pltpu.make_async_copy(
            bufs[b], out_hbm.at[pl.ds(base + j, 1)], ssems[b]).wait()


def kernel(prefix, emb_table):
    idx = prefix.astype(jnp.int32).reshape(NW, BPW, 1)
    out = _sc_gather(idx, emb_table)
    return out.reshape(B, S, D)
